# Initial kernel scaffold; baseline (speedup 1.0000x reference)
import jax
import jax.numpy as jnp
from jax.experimental import pallas as pl


def kernel(prods, atom_types, target_angles, atom_types_probabilities, valency_weight):
    raise NotImplementedError("write your pallas kernel here")


# speedup vs baseline: 17.2835x; 17.2835x over previous
import functools
import math

import numpy as np
import jax
import jax.numpy as jnp
from jax import lax
from jax.experimental import pallas as pl
from jax.experimental.pallas import tpu as pltpu
from jax.experimental.pallas import tpu_sc as plsc

_N = 8388608
_T = 16
_B = 1801
_HB = _T * _B
_NC = 2
_NS = 16
_NW = _NC * _NS
_PER_W = _N // _NW
_CHUNK = 16384
_NCHUNK = _PER_W // _CHUNK
_SHIFT = 14

_CLIP_MAX = np.float32(1.0 - 1e-6)


def _build_tables():
  delta = math.pi / 1800.0
  j = np.arange(900, dtype=np.float64)
  wb = (1.0 - np.cos((j + 0.5) * delta)).astype(np.float32)
  w_min = np.float32(1.0) - _CLIP_MAX
  w_max = np.float32(1.0)
  c_min = int(np.float32(w_min).view(np.uint32)) >> _SHIFT
  c_max = int(np.float32(w_max).view(np.uint32)) >> _SHIFT
  cells = np.arange(c_min, c_max + 1, dtype=np.uint64)
  hi_bits = (((cells + 1) << _SHIFT) - 1).astype(np.uint32)
  w_hi = np.minimum(hi_bits.view(np.float32), w_max)
  lo_bits = (cells << _SHIFT).astype(np.uint32)
  w_lo = np.maximum(lo_bits.view(np.float32), w_min)
  k_hi = np.searchsorted(wb, w_hi, side="right").astype(np.int32)
  k_lo = np.searchsorted(wb, w_lo, side="right").astype(np.int32)
  if int((k_hi - k_lo).max()) > 1:
    raise AssertionError("bin table cell spans more than two bins")
  bt = np.zeros((901,), np.float32)
  bt[1:] = wb

  def pad16(a):
    return np.pad(a, (0, (-len(a)) % 16))

  return pad16(k_hi), pad16(bt), c_min


_TBL_NP, _BT_NP, _C_MIN = _build_tables()
_TBL_N = _TBL_NP.shape[0]
_BT_N = _BT_NP.shape[0]

_mesh = plsc.VectorSubcoreMesh(core_axis_name="c", subcore_axis_name="s")


@functools.partial(
    pl.kernel,
    out_type=jax.ShapeDtypeStruct((_NW * _HB,), jnp.float32),
    mesh=_mesh,
    compiler_params=pltpu.CompilerParams(needs_layout_passes=False),
    scratch_types=[
        pltpu.VMEM((_CHUNK,), jnp.float32),
        pltpu.VMEM((_CHUNK,), jnp.float32),
        pltpu.VMEM((_CHUNK,), jnp.int32),
        pltpu.VMEM((_CHUNK,), jnp.int32),
        pltpu.VMEM((_HB,), jnp.float32),
        pltpu.VMEM((_TBL_N,), jnp.int32),
        pltpu.VMEM((_BT_N,), jnp.float32),
        pltpu.SemaphoreType.DMA,
        pltpu.SemaphoreType.DMA,
        pltpu.SemaphoreType.DMA,
        pltpu.SemaphoreType.DMA,
    ],
)
def _hist_sc(prods_hbm, at_hbm, tbl_hbm, bt_hbm, out_hbm,
             pb0, pb1, ab0, ab1, hist, tbl, bt, sp0, sp1, sa0, sa1):
  wid = lax.axis_index("s") * _NC + lax.axis_index("c")
  base = wid * _PER_W

  pltpu.sync_copy(tbl_hbm, tbl)
  pltpu.sync_copy(bt_hbm, bt)

  def zbody(i, c):
    hist[pl.ds(i * 16, 16)] = jnp.zeros((16,), jnp.float32)
    return c

  lax.fori_loop(0, _HB // 16, zbody, 0)

  pbufs = (pb0, pb1)
  abufs = (ab0, ab1)
  psems = (sp0, sp1)
  asems = (sa0, sa1)

  def start(g):
    slot = g % 2
    off = base + g * _CHUNK
    hp = pltpu.async_copy(prods_hbm.at[pl.ds(off, _CHUNK)], pbufs[slot],
                          psems[slot])
    ha = pltpu.async_copy(at_hbm.at[pl.ds(off, _CHUNK)], abufs[slot],
                          asems[slot])
    return hp, ha

  handles = [start(0), None]
  for g in range(_NCHUNK):
    slot = g % 2
    if g + 1 < _NCHUNK:
      handles[1 - slot] = start(g + 1)
    hp, ha = handles[slot]
    hp.wait()
    ha.wait()
    pb_r = pbufs[slot]
    ab_r = abufs[slot]

    def body(i, c, pb_r=pb_r, ab_r=ab_r):
      off = i * 16
      p = pb_r[pl.ds(off, 16)]
      a = ab_r[pl.ds(off, 16)]
      p = jnp.minimum(jnp.maximum(p, 0.0), _CLIP_MAX)
      w = 1.0 - p
      u = lax.bitcast_convert_type(w, jnp.int32)
      cell = lax.shift_right_logical(u, _SHIFT) - _C_MIN
      k = plsc.load_gather(tbl, [cell])
      lb = plsc.load_gather(bt, [k])
      binv = k - jnp.where(w < lb, 1, 0)
      idx = a * _B + binv
      cnt, last = plsc.scan_count(idx)
      plsc.addupdate_scatter(hist, [idx], cnt.astype(jnp.float32), mask=last)
      return c

    lax.fori_loop(0, _CHUNK // 16, body, 0)

  pltpu.sync_copy(hist, out_hbm.at[pl.ds(wid * _HB, _HB)])


def _tail_body(h_ref, t_ref, pr_ref, vw_ref, o_ref):
  g = h_ref[0:_T, :]
  for wkr in range(1, _NW):
    g = g + h_ref[wkr * _T:(wkr + 1) * _T, :]
  s = jnp.sum(g, axis=1, keepdims=True)
  s = jnp.where(s == 0.0, 1.0, s)
  g = g / s
  d = g - t_ref[...]
  row = lax.broadcasted_iota(jnp.int32, (_B, _B), 0)
  col = lax.broadcasted_iota(jnp.int32, (_B, _B), 1)
  tri = jnp.where(row <= col, 1.0, 0.0)
  cs = jnp.dot(d, tri, preferred_element_type=jnp.float32)
  w1 = jnp.sum(jnp.abs(cs), axis=1, keepdims=True) / 10.0
  pv = pr_ref[...] * vw_ref[...]
  total = jnp.sum(w1 * pv) / (jnp.sum(pv) + 1e-5)
  o_ref[0, 0] = total


_tail = pl.pallas_call(
    _tail_body,
    out_shape=jax.ShapeDtypeStruct((1, 1), jnp.float32),
    out_specs=pl.BlockSpec(memory_space=pltpu.SMEM),
)


def kernel(prods, atom_types, target_angles, atom_types_probabilities,
           valency_weight):
  hist = _hist_sc(prods, atom_types, jnp.asarray(_TBL_NP), jnp.asarray(_BT_NP))
  h2 = hist.reshape(_NW * _T, _B)
  out = _tail(h2, target_angles,
              atom_types_probabilities.reshape(_T, 1),
              valency_weight.reshape(_T, 1))
  return out.reshape(())


# speedup vs baseline: 104.6577x; 6.0554x over previous
import functools
import math

import numpy as np
import jax
import jax.numpy as jnp
from jax import lax
from jax.experimental import pallas as pl
from jax.experimental.pallas import tpu as pltpu
from jax.experimental.pallas import tpu_sc as plsc

_N = 8388608
_T = 16
_B = 1801
_HB = _T * _B
_NC = 2
_NS = 16
_NW = _NC * _NS
_PER_W = _N // _NW
_CHUNK = 16384
_NCHUNK = _PER_W // _CHUNK
_SHIFT = 14

_CLIP_MAX = np.float32(1.0 - 1e-6)


def _build_tables():
  delta = math.pi / 1800.0
  j = np.arange(900, dtype=np.float64)
  wb = (1.0 - np.cos((j + 0.5) * delta)).astype(np.float32)
  w_min = np.float32(1.0) - _CLIP_MAX
  w_max = np.float32(1.0)
  c_min = int(np.float32(w_min).view(np.uint32)) >> _SHIFT
  c_max = int(np.float32(w_max).view(np.uint32)) >> _SHIFT
  cells = np.arange(c_min, c_max + 1, dtype=np.uint64)
  hi_bits = (((cells + 1) << _SHIFT) - 1).astype(np.uint32)
  w_hi = np.minimum(hi_bits.view(np.float32), w_max)
  lo_bits = (cells << _SHIFT).astype(np.uint32)
  w_lo = np.maximum(lo_bits.view(np.float32), w_min)
  k_hi = np.searchsorted(wb, w_hi, side="right").astype(np.int64)
  k_lo = np.searchsorted(wb, w_lo, side="right").astype(np.int64)
  if int((k_hi - k_lo).max()) > 1:
    raise AssertionError("bin table cell spans more than two bins")
  bt = np.zeros((901,), np.float32)
  bt[1:] = wb
  bd_bits = bt[k_hi].view(np.uint32).astype(np.int64)
  packed = ((bd_bits & ~np.int64(0x3FF)) | k_hi).astype(np.uint32)

  def pad16(a):
    return np.pad(a, (0, (-len(a)) % 16))

  return pad16(packed).view(np.int32), c_min


_TBL_NP, _C_MIN = _build_tables()
_TBL_N = _TBL_NP.shape[0]

_mesh = plsc.VectorSubcoreMesh(core_axis_name="c", subcore_axis_name="s")


@functools.partial(
    pl.kernel,
    out_type=jax.ShapeDtypeStruct((_NW * _HB,), jnp.float32),
    mesh=_mesh,
    compiler_params=pltpu.CompilerParams(needs_layout_passes=False),
    scratch_types=[
        pltpu.VMEM((_CHUNK,), jnp.float32),
        pltpu.VMEM((_CHUNK,), jnp.float32),
        pltpu.VMEM((_CHUNK,), jnp.int32),
        pltpu.VMEM((_CHUNK,), jnp.int32),
        pltpu.VMEM((_HB,), jnp.float32),
        pltpu.VMEM((_TBL_N,), jnp.int32),
        pltpu.SemaphoreType.DMA,
        pltpu.SemaphoreType.DMA,
        pltpu.SemaphoreType.DMA,
        pltpu.SemaphoreType.DMA,
    ],
)
def _hist_sc(prods_hbm, at_hbm, tbl_hbm, out_hbm,
             pb0, pb1, ab0, ab1, hist, tbl, sp0, sp1, sa0, sa1):
  wid = lax.axis_index("s") * _NC + lax.axis_index("c")
  base = wid * _PER_W

  pltpu.sync_copy(tbl_hbm, tbl)

  @functools.partial(plsc.parallel_loop, 0, _HB // 16, unroll=8)
  def _(i):
    hist[pl.ds(i * 16, 16)] = jnp.zeros((16,), jnp.float32)

  pbufs = (pb0, pb1)
  abufs = (ab0, ab1)
  psems = (sp0, sp1)
  asems = (sa0, sa1)

  def start(g):
    slot = g % 2
    off = base + g * _CHUNK
    hp = pltpu.async_copy(prods_hbm.at[pl.ds(off, _CHUNK)], pbufs[slot],
                          psems[slot])
    ha = pltpu.async_copy(at_hbm.at[pl.ds(off, _CHUNK)], abufs[slot],
                          asems[slot])
    return hp, ha

  handles = [start(0), None]
  for g in range(_NCHUNK):
    slot = g % 2
    if g + 1 < _NCHUNK:
      handles[1 - slot] = start(g + 1)
    hp, ha = handles[slot]
    hp.wait()
    ha.wait()
    pb_r = pbufs[slot]
    ab_r = abufs[slot]

    def body(i, pb_r=pb_r, ab_r=ab_r):
      off = i * 16
      p = pb_r[pl.ds(off, 16)]
      a = ab_r[pl.ds(off, 16)]
      p = jnp.minimum(p, _CLIP_MAX)
      w = 1.0 - p
      u = lax.bitcast_convert_type(w, jnp.int32)
      cell = lax.shift_right_logical(u, _SHIFT) - _C_MIN
      t = plsc.load_gather(tbl, [cell])
      k = t & 0x3FF
      lb = t & ~0x3FF
      binv = k - jnp.where(u < lb, 1, 0)
      idx = a * _B + binv
      cnt, last = plsc.scan_count(idx)
      plsc.addupdate_scatter(hist, [idx], cnt.astype(jnp.float32), mask=last)

    plsc.parallel_loop(0, _CHUNK // 16, unroll=8)(body)

  pltpu.sync_copy(hist, out_hbm.at[pl.ds(wid * _HB, _HB)])


def _tail_body(h_ref, t_ref, pr_ref, vw_ref, o_ref):
  g = h_ref[0:_T, :]
  for wkr in range(1, _NW):
    g = g + h_ref[wkr * _T:(wkr + 1) * _T, :]
  s = jnp.sum(g, axis=1, keepdims=True)
  s = jnp.where(s == 0.0, 1.0, s)
  g = g / s
  d = g - t_ref[...]
  row = lax.broadcasted_iota(jnp.int32, (_B, _B), 0)
  col = lax.broadcasted_iota(jnp.int32, (_B, _B), 1)
  tri = jnp.where(row <= col, 1.0, 0.0)
  cs = jnp.dot(d, tri, preferred_element_type=jnp.float32)
  w1 = jnp.sum(jnp.abs(cs), axis=1, keepdims=True) / 10.0
  pv = pr_ref[...] * vw_ref[...]
  total = jnp.sum(w1 * pv) / (jnp.sum(pv) + 1e-5)
  o_ref[0, 0] = total


_tail = pl.pallas_call(
    _tail_body,
    out_shape=jax.ShapeDtypeStruct((1, 1), jnp.float32),
    out_specs=pl.BlockSpec(memory_space=pltpu.SMEM),
)


def kernel(prods, atom_types, target_angles, atom_types_probabilities,
           valency_weight):
  hist = _hist_sc(prods, atom_types, jnp.asarray(_TBL_NP))
  h2 = hist.reshape(_NW * _T, _B)
  out = _tail(h2, target_angles,
              atom_types_probabilities.reshape(_T, 1),
              valency_weight.reshape(_T, 1))
  return out.reshape(())
